# baseline (speedup 1.0000x reference)
import jax
import jax.numpy as jnp
from jax import lax
from jax.experimental import pallas as pl
from jax.experimental.pallas import tpu as pltpu
from jax.experimental.pallas import tpu_sc as plsc

DIM = 16
B = 4096
N_NB = 20
B1 = B * N_NB

NC, NS, L = 2, 16, 16
NW = NC * NS
SEG = 128
C1 = 128
NCHUNK1 = B1 // (NW * C1)
NSEG1 = C1 * N_NB // SEG
TROW1 = B1 // (NW * SEG)
ROWS2 = B // NW
NSEG2 = ROWS2 * N_NB // SEG


def _rsqrt_newton(x):
    i = lax.bitcast_convert_type(x, jnp.int32)
    i = jnp.int32(0x5F3759DF) - lax.shift_right_arithmetic(i, 1)
    y = lax.bitcast_convert_type(i, jnp.float32)
    for _ in range(3):
        y = y * (1.5 - 0.5 * x * y * y)
    return y


def _fm_rows(rows_v, t_rows_v, out_v, n_rows):
    def row_body(r, carry):
        z = jnp.zeros((L,), jnp.float32)
        accs = [z, z, z, z]
        sqs = [z, z, z, z]
        for j in range(N_NB):
            v = rows_v[r * N_NB + j, :]
            accs[j % 4] = accs[j % 4] + v
            sqs[j % 4] = sqs[j % 4] + v * v
        acc = (accs[0] + accs[1]) + (accs[2] + accs[3])
        sq = (sqs[0] + sqs[1]) + (sqs[2] + sqs[3])
        out_v[r, :] = acc * acc - sq + t_rows_v[r, :]
        return carry

    lax.fori_loop(0, n_rows, row_body, 0)


def _hop1_body(nbr3d, tidx3d, table, agg_out, idx_v, t_idx_v, rows_v,
               t_rows_v, agg_v, sem0, sem1, t_sem0, t_sem1):
    w = lax.axis_index("s") * NC + lax.axis_index("c")
    sems = [sem0, sem1]
    t_sems = [t_sem0, t_sem1]
    pltpu.sync_copy(tidx3d.at[w], t_idx_v)

    def fire(c, b):
        pltpu.sync_copy(nbr3d.at[w, pl.ds(c * NSEG1, NSEG1)], idx_v.at[b])
        for s in range(NSEG1):
            pltpu.async_copy(table.at[idx_v.at[b, s]],
                             rows_v.at[b, pl.ds(s * SEG, SEG)], sems[b])
        pltpu.async_copy(table.at[t_idx_v.at[c]], t_rows_v.at[b], t_sems[b])

    def drain(b):
        pltpu.make_async_copy(table.at[pl.ds(0, C1 * N_NB)],
                              rows_v.at[b], sems[b]).wait()
        pltpu.make_async_copy(table.at[pl.ds(0, C1)],
                              t_rows_v.at[b], t_sems[b]).wait()

    def compute(c, b):
        _fm_rows(rows_v.at[b], t_rows_v.at[b], agg_v, C1)
        base = w * (NCHUNK1 * C1) + c * C1
        pltpu.sync_copy(agg_v, agg_out.at[pl.ds(base, C1)])

    fire(0, 0)

    def pair_body(p, carry):
        c0 = 2 * p
        fire(c0 + 1, 1)
        drain(0)
        compute(c0, 0)

        @pl.when(p < NCHUNK1 // 2 - 1)
        def _():
            fire(c0 + 2, 0)

        drain(1)
        compute(c0 + 1, 1)
        return carry

    lax.fori_loop(0, NCHUNK1 // 2, pair_body, 0)


def _hop2_body(nbr3d, tidx3d, uidx3d, agg, table, users_t, out,
               idx_v, t_idx_v, u_idx_v, rows_v, t_rows_v, u_t_v,
               items_v, out_v, sem, t_sem):
    w = lax.axis_index("s") * NC + lax.axis_index("c")
    pltpu.sync_copy(nbr3d.at[w], idx_v)
    pltpu.sync_copy(tidx3d.at[w], t_idx_v)
    pltpu.sync_copy(uidx3d.at[w], u_idx_v)
    copies = []
    for s in range(NSEG2):
        copies.append(pltpu.async_copy(
            agg.at[idx_v.at[s]], rows_v.at[pl.ds(s * SEG, SEG)], sem))
    t_copy = pltpu.async_copy(table.at[t_idx_v.at[0]], t_rows_v, t_sem)
    u_copies = []
    for d in range(DIM):
        u_copies.append(pltpu.async_copy(
            users_t.at[d].at[u_idx_v.at[0]], u_t_v.at[d], t_sem))
    for cp in copies:
        cp.wait()
    t_copy.wait()
    for cp in u_copies:
        cp.wait()

    _fm_rows(rows_v, t_rows_v, items_v, ROWS2)

    for g in range(ROWS2 // L):
        rows16 = lax.iota(jnp.int32, L) + jnp.int32(g * L)
        uv = jnp.zeros((L,), jnp.float32)
        nsq = jnp.zeros((L,), jnp.float32)
        for d in range(DIM):
            dsplat = jnp.full((L,), d, jnp.int32)
            uc = u_t_v[d, pl.ds(g * L, L)]
            ic = plsc.load_gather(items_v, [rows16, dsplat])
            uv = uv + uc * ic
            nsq = nsq + uc * uc
        scale = jnp.minimum(1.0, _rsqrt_newton(jnp.maximum(nsq, 1e-14)))
        x = uv * scale
        out_v[pl.ds(g * L, L)] = 1.0 / (1.0 + jnp.exp(-x))

    pltpu.sync_copy(out_v, out.at[pl.ds(w * ROWS2, ROWS2)])


def kernel(u, hop1_index, hop1_neighbors, hop2_index, hop2_neighbors,
           entitys_table, users_table):
    mesh = plsc.VectorSubcoreMesh(core_axis_name="c", subcore_axis_name="s")

    nbr1_3d = hop1_neighbors.reshape(NW, NCHUNK1 * NSEG1, SEG)
    tidx1_3d = hop1_index.reshape(NW, TROW1, SEG)
    nbr2_3d = hop2_neighbors.reshape(NW, NSEG2, SEG)
    tidx2_3d = hop2_index.reshape(NW, 1, SEG)
    u_3d = u.reshape(NW, 1, SEG)

    hop1 = pl.kernel(
        _hop1_body, mesh=mesh,
        compiler_params=pltpu.CompilerParams(
            use_tc_tiling_on_sc=False, needs_layout_passes=False),
        out_type=jax.ShapeDtypeStruct((B1, DIM), jnp.float32),
        scratch_types=[
            pltpu.VMEM((2, NSEG1, SEG), jnp.int32),
            pltpu.VMEM((TROW1, SEG), jnp.int32),
            pltpu.VMEM((2, C1 * N_NB, DIM), jnp.float32),
            pltpu.VMEM((2, C1, DIM), jnp.float32),
            pltpu.VMEM((C1, DIM), jnp.float32),
            pltpu.SemaphoreType.DMA,
            pltpu.SemaphoreType.DMA,
            pltpu.SemaphoreType.DMA,
            pltpu.SemaphoreType.DMA,
        ],
    )
    agg1 = hop1(nbr1_3d, tidx1_3d, entitys_table)

    hop2 = pl.kernel(
        _hop2_body, mesh=mesh,
        compiler_params=pltpu.CompilerParams(
            use_tc_tiling_on_sc=False, needs_layout_passes=False),
        out_type=jax.ShapeDtypeStruct((B,), jnp.float32),
        scratch_types=[
            pltpu.VMEM((NSEG2, SEG), jnp.int32),
            pltpu.VMEM((1, SEG), jnp.int32),
            pltpu.VMEM((1, SEG), jnp.int32),
            pltpu.VMEM((ROWS2 * N_NB, DIM), jnp.float32),
            pltpu.VMEM((ROWS2, DIM), jnp.float32),
            pltpu.VMEM((DIM, SEG), jnp.float32),
            pltpu.VMEM((ROWS2, DIM), jnp.float32),
            pltpu.VMEM((ROWS2,), jnp.float32),
            pltpu.SemaphoreType.DMA,
            pltpu.SemaphoreType.DMA,
        ],
    )
    logit = hop2(nbr2_3d, tidx2_3d, u_3d, agg1, entitys_table,
                 users_table.T)
    return logit


# speedup vs baseline: 2.3533x; 2.3533x over previous
import jax
import jax.numpy as jnp
from jax import lax
from jax.experimental import pallas as pl
from jax.experimental.pallas import tpu as pltpu
from jax.experimental.pallas import tpu_sc as plsc

DIM = 16
B = 4096
N_NB = 20
B1 = B * N_NB

NC, NS, L = 2, 16, 16
NW = NC * NS
SEG = 128
C1 = 128
NCHUNK1 = B1 // (NW * C1)
NSEG1 = C1 * N_NB // SEG
TROW1 = B1 // (NW * SEG)
ROWS2 = B // NW
NSEG2 = ROWS2 * N_NB // SEG


def _rsqrt_newton(x):
    i = lax.bitcast_convert_type(x, jnp.int32)
    i = jnp.int32(0x5F3759DF) - lax.shift_right_arithmetic(i, 1)
    y = lax.bitcast_convert_type(i, jnp.float32)
    for _ in range(3):
        y = y * (1.5 - 0.5 * x * y * y)
    return y


def _fm_rows(rows_v, t_rows_v, out_v, n_rows):
    def row_body(r, carry):
        z = jnp.zeros((L,), jnp.float32)
        accs = [z, z, z, z]
        sqs = [z, z, z, z]
        for j in range(N_NB):
            v = rows_v[r * N_NB + j, :]
            accs[j % 4] = accs[j % 4] + v
            sqs[j % 4] = sqs[j % 4] + v * v
        acc = (accs[0] + accs[1]) + (accs[2] + accs[3])
        sq = (sqs[0] + sqs[1]) + (sqs[2] + sqs[3])
        out_v[r, :] = acc * acc - sq + t_rows_v[r, :]
        return carry

    lax.fori_loop(0, n_rows, row_body, 0)


def _hop1_body(nbr3d, tidx3d, table, agg_out, idx_v, t_idx_v, rows_v,
               t_rows_v, agg_v, sem0, sem1, t_sem0, t_sem1):
    w = lax.axis_index("s") * NC + lax.axis_index("c")
    sems = [sem0, sem1]
    t_sems = [t_sem0, t_sem1]
    pltpu.sync_copy(tidx3d.at[w], t_idx_v)

    def fire(c, b):
        pltpu.sync_copy(nbr3d.at[w, pl.ds(c * NSEG1, NSEG1)], idx_v.at[b])
        for s in range(NSEG1):
            pltpu.async_copy(table.at[idx_v.at[b, s]],
                             rows_v.at[b, pl.ds(s * SEG, SEG)], sems[b])
        pltpu.async_copy(table.at[t_idx_v.at[c]], t_rows_v.at[b], t_sems[b])

    def drain(b):
        pltpu.make_async_copy(table.at[pl.ds(0, C1 * N_NB)],
                              rows_v.at[b], sems[b]).wait()
        pltpu.make_async_copy(table.at[pl.ds(0, C1)],
                              t_rows_v.at[b], t_sems[b]).wait()

    def compute(c, b):
        _fm_rows(rows_v.at[b], t_rows_v.at[b], agg_v, C1)
        base = w * (NCHUNK1 * C1) + c * C1
        pltpu.sync_copy(agg_v, agg_out.at[pl.ds(base, C1)])

    fire(0, 0)

    def pair_body(p, carry):
        c0 = 2 * p
        fire(c0 + 1, 1)
        drain(0)
        compute(c0, 0)

        @pl.when(p < NCHUNK1 // 2 - 1)
        def _():
            fire(c0 + 2, 0)

        drain(1)
        compute(c0 + 1, 1)
        return carry

    lax.fori_loop(0, NCHUNK1 // 2, pair_body, 0)


def _ufetch_body(uidx3d, users, out, u_idx_v, g_v, packed_v, sem):
    w = lax.axis_index("s") * NC + lax.axis_index("c")
    pltpu.sync_copy(uidx3d.at[w], u_idx_v)
    for grp in range(ROWS2 // 8):
        copies = []
        for k in range(8):
            i = grp * 8 + k
            ui = u_idx_v[0, pl.ds((i // L) * L, L)][i % L]
            e8 = pl.multiple_of((ui // 8) * 8, 8)
            copies.append(pltpu.async_copy(
                users.at[pl.ds(e8, 8)], g_v.at[k], sem))
        for cp in copies:
            cp.wait()
        for k in range(8):
            i = grp * 8 + k
            ui = u_idx_v[0, pl.ds((i // L) * L, L)][i % L]
            packed_v[pl.ds(i * DIM, DIM)] = g_v[k, ui % 8, :]
    pltpu.sync_copy(packed_v, out.at[pl.ds(w * ROWS2 * DIM, ROWS2 * DIM)])


def _hop2_body(nbr3d, tidx3d, uidx3d, agg, table, users, out,
               idx_v, t_idx_v, u_idx_v, rows_v, t_rows_v, u_rows_v,
               items_v, out_v, sem, t_sem):
    w = lax.axis_index("s") * NC + lax.axis_index("c")
    pltpu.sync_copy(nbr3d.at[w], idx_v)
    pltpu.sync_copy(tidx3d.at[w], t_idx_v)
    pltpu.sync_copy(uidx3d.at[w], u_idx_v)
    copies = []
    for s in range(NSEG2):
        copies.append(pltpu.async_copy(
            agg.at[idx_v.at[s]], rows_v.at[pl.ds(s * SEG, SEG)], sem))
    t_copy = pltpu.async_copy(table.at[t_idx_v.at[0]], t_rows_v, t_sem)
    u_copy = pltpu.async_copy(users.at[pl.ds(w * ROWS2, ROWS2)], u_rows_v,
                              t_sem)
    for cp in copies:
        cp.wait()
    t_copy.wait()
    u_copy.wait()

    _fm_rows(rows_v, t_rows_v, items_v, ROWS2)

    for g in range(ROWS2 // L):
        rows16 = lax.iota(jnp.int32, L) + jnp.int32(g * L)
        uv = jnp.zeros((L,), jnp.float32)
        nsq = jnp.zeros((L,), jnp.float32)
        for d in range(DIM):
            dsplat = jnp.full((L,), d, jnp.int32)
            uc = plsc.load_gather(u_rows_v, [rows16, dsplat])
            ic = plsc.load_gather(items_v, [rows16, dsplat])
            uv = uv + uc * ic
            nsq = nsq + uc * uc
        scale = jnp.minimum(1.0, _rsqrt_newton(jnp.maximum(nsq, 1e-14)))
        x = uv * scale
        out_v[pl.ds(g * L, L)] = 1.0 / (1.0 + jnp.exp(-x))

    pltpu.sync_copy(out_v, out.at[pl.ds(w * ROWS2, ROWS2)])


def kernel(u, hop1_index, hop1_neighbors, hop2_index, hop2_neighbors,
           entitys_table, users_table):
    mesh = plsc.VectorSubcoreMesh(core_axis_name="c", subcore_axis_name="s")

    nbr1_3d = hop1_neighbors.reshape(NW, NCHUNK1 * NSEG1, SEG)
    tidx1_3d = hop1_index.reshape(NW, TROW1, SEG)
    nbr2_3d = hop2_neighbors.reshape(NW, NSEG2, SEG)
    tidx2_3d = hop2_index.reshape(NW, 1, SEG)
    u_3d = u.reshape(NW, 1, SEG)

    hop1 = pl.kernel(
        _hop1_body, mesh=mesh,
        compiler_params=pltpu.CompilerParams(
            use_tc_tiling_on_sc=False, needs_layout_passes=False),
        out_type=jax.ShapeDtypeStruct((B1, DIM), jnp.float32),
        scratch_types=[
            pltpu.VMEM((2, NSEG1, SEG), jnp.int32),
            pltpu.VMEM((TROW1, SEG), jnp.int32),
            pltpu.VMEM((2, C1 * N_NB, DIM), jnp.float32),
            pltpu.VMEM((2, C1, DIM), jnp.float32),
            pltpu.VMEM((C1, DIM), jnp.float32),
            pltpu.SemaphoreType.DMA,
            pltpu.SemaphoreType.DMA,
            pltpu.SemaphoreType.DMA,
            pltpu.SemaphoreType.DMA,
        ],
    )
    agg1 = hop1(nbr1_3d, tidx1_3d, entitys_table)

    ufetch = pl.kernel(
        _ufetch_body, mesh=mesh,
        out_type=jax.ShapeDtypeStruct((B * DIM,), jnp.float32),
        scratch_types=[
            pltpu.VMEM((1, SEG), jnp.int32),
            pltpu.VMEM((8, 8, DIM), jnp.float32),
            pltpu.VMEM((ROWS2 * DIM,), jnp.float32),
            pltpu.SemaphoreType.DMA,
        ],
    )
    users_packed = ufetch(u_3d, users_table).reshape(B, DIM)

    hop2 = pl.kernel(
        _hop2_body, mesh=mesh,
        compiler_params=pltpu.CompilerParams(
            use_tc_tiling_on_sc=False, needs_layout_passes=False),
        out_type=jax.ShapeDtypeStruct((B,), jnp.float32),
        scratch_types=[
            pltpu.VMEM((NSEG2, SEG), jnp.int32),
            pltpu.VMEM((1, SEG), jnp.int32),
            pltpu.VMEM((1, SEG), jnp.int32),
            pltpu.VMEM((ROWS2 * N_NB, DIM), jnp.float32),
            pltpu.VMEM((ROWS2, DIM), jnp.float32),
            pltpu.VMEM((ROWS2, DIM), jnp.float32),
            pltpu.VMEM((ROWS2, DIM), jnp.float32),
            pltpu.VMEM((ROWS2,), jnp.float32),
            pltpu.SemaphoreType.DMA,
            pltpu.SemaphoreType.DMA,
        ],
    )
    logit = hop2(nbr2_3d, tidx2_3d, u_3d, agg1, entitys_table,
                 users_packed)
    return logit
